# Initial kernel scaffold; baseline (speedup 1.0000x reference)
import jax
import jax.numpy as jnp
from jax.experimental import pallas as pl


def kernel(x, edge_index, Wl0, bl0, Wr0, Wl1, bl1, Wr1):
    raise NotImplementedError("write your pallas kernel here")


# speedup vs baseline: 6.0861x; 6.0861x over previous
import jax
import jax.numpy as jnp
from jax import lax
from jax.experimental import pallas as pl
from jax.experimental.pallas import tpu as pltpu
from jax.experimental.pallas import tpu_sc as plsc

N = 10000
E = 320000
D = 128
CW = 16
SHIFT = 14

NC = 2
NS = 16
NW = NC * NS
EPW = E // NW
C = 80
CHUNKS = EPW // C
RPT = 624
RLAST = N - (NS - 1) * RPT
WCH = 80


def _agg_body(x_hbm, eidx_hbm, outp_hbm, outc_hbm,
              idx_v, src_v, dst_v, rows_v, ones_v, obuf_v, cbuf_v,
              acc_sh, cnt_sh, sem):
    cid = lax.axis_index("c")
    sid = lax.axis_index("s")
    wid = sid * NC + cid
    zbase = sid * RPT
    base_e = wid * EPW
    last = sid == NS - 1

    z16 = jnp.zeros((16,), jnp.float32)
    o16 = jnp.ones((16,), jnp.float32)

    @pl.loop(0, WCH)
    def _zrow(r):
        for j in range(D // 16):
            obuf_v[r, pl.ds(j * 16, 16)] = z16
        cbuf_v[r, :] = z16
        ones_v[r, :] = o16

    for k in range(7):
        pltpu.sync_copy(obuf_v, acc_sh.at[pl.ds(zbase + k * WCH, WCH)])
        pltpu.sync_copy(cbuf_v, cnt_sh.at[pl.ds(zbase + k * WCH, WCH)])

    @pl.when(~last)
    def _():
        pltpu.sync_copy(obuf_v.at[pl.ds(0, RPT - 7 * WCH)],
                        acc_sh.at[pl.ds(zbase + 7 * WCH, RPT - 7 * WCH)])
        pltpu.sync_copy(cbuf_v.at[pl.ds(0, RPT - 7 * WCH)],
                        cnt_sh.at[pl.ds(zbase + 7 * WCH, RPT - 7 * WCH)])

    @pl.when(last)
    def _():
        pltpu.sync_copy(obuf_v, acc_sh.at[pl.ds(zbase + 7 * WCH, WCH)])
        pltpu.sync_copy(cbuf_v, cnt_sh.at[pl.ds(zbase + 7 * WCH, WCH)])

    plsc.subcore_barrier()

    @pl.loop(0, CHUNKS)
    def _chunk(i):
        off = base_e + i * C
        pltpu.sync_copy(eidx_hbm.at[pl.ds(off, C)], idx_v)
        for k in range(C // 16):
            w = idx_v[pl.ds(k * 16, 16)]
            src_v[pl.ds(k * 16, 16)] = lax.shift_right_logical(w, SHIFT)
            dst_v[0, pl.ds(k * 16, 16)] = lax.bitwise_and(w, (1 << SHIFT) - 1)
        pltpu.async_copy(x_hbm.at[src_v], rows_v, sem).wait()
        pltpu.sync_copy(rows_v, acc_sh.at[dst_v.at[0]], add=True)
        pltpu.sync_copy(ones_v, cnt_sh.at[dst_v.at[0]], add=True)

    plsc.subcore_barrier()

    def out_chunk(rbase, nrows):
        pltpu.sync_copy(acc_sh.at[pl.ds(rbase, nrows)],
                        obuf_v.at[pl.ds(0, nrows)])
        pltpu.sync_copy(obuf_v.at[pl.ds(0, nrows)],
                        outp_hbm.at[cid, pl.ds(rbase, nrows)])
        pltpu.sync_copy(cnt_sh.at[pl.ds(rbase, nrows)],
                        cbuf_v.at[pl.ds(0, nrows)])
        pltpu.sync_copy(cbuf_v.at[pl.ds(0, nrows)],
                        outc_hbm.at[cid, pl.ds(rbase, nrows)])

    for k in range(7):
        out_chunk(zbase + k * WCH, WCH)

    @pl.when(~last)
    def _():
        out_chunk(zbase + 7 * WCH, RPT - 7 * WCH)

    @pl.when(last)
    def _():
        out_chunk(zbase + 7 * WCH, WCH)


_MESH = plsc.VectorSubcoreMesh(core_axis_name="c", subcore_axis_name="s")
_AGG = pl.kernel(
    _agg_body,
    out_type=[
        jax.ShapeDtypeStruct((NC, N, D), jnp.float32),
        jax.ShapeDtypeStruct((NC, N, CW), jnp.float32),
    ],
    mesh=_MESH,
    compiler_params=pltpu.CompilerParams(use_tc_tiling_on_sc=False),
    scratch_types=[
        pltpu.VMEM((C,), jnp.int32),
        pltpu.VMEM((C,), jnp.int32),
        pltpu.VMEM((1, C), jnp.int32),
        pltpu.VMEM((C, D), jnp.float32),
        pltpu.VMEM((WCH, CW), jnp.float32),
        pltpu.VMEM((WCH, D), jnp.float32),
        pltpu.VMEM((WCH, CW), jnp.float32),
        pltpu.VMEM_SHARED((N, D), jnp.float32),
        pltpu.VMEM_SHARED((N, CW), jnp.float32),
        pltpu.SemaphoreType.DMA,
    ],
)


R = 2000


def _tc_layer_body(p_ref, c_ref, x_ref, wlT_ref, bl_ref, wrT_ref, th_ref,
                   o_ref):
    s = p_ref[0] + p_ref[1]
    cnt = c_ref[0, :, 0:1] + c_ref[1, :, 0:1]
    mean = s / jnp.maximum(cnt, 1.0)
    h = (jnp.dot(mean, wlT_ref[...], preferred_element_type=jnp.float32)
         + bl_ref[...]
         + jnp.dot(x_ref[...], wrT_ref[...], preferred_element_type=jnp.float32))
    o_ref[...] = jnp.maximum(h, th_ref[0, 0])


def _tc_layer(p, c, x, wlT, bl2, wrT, thresh):
    grid = (N // R,)
    in_specs = [
        pl.BlockSpec((NC, R, D), lambda i: (0, i, 0)),
        pl.BlockSpec((NC, R, CW), lambda i: (0, i, 0)),
        pl.BlockSpec((R, D), lambda i: (i, 0)),
        pl.BlockSpec((D, D), lambda i: (0, 0)),
        pl.BlockSpec((1, D), lambda i: (0, 0)),
        pl.BlockSpec((D, D), lambda i: (0, 0)),
        pl.BlockSpec((1, 1), lambda i: (0, 0)),
    ]
    return pl.pallas_call(
        _tc_layer_body,
        grid=grid,
        in_specs=in_specs,
        out_specs=pl.BlockSpec((R, D), lambda i: (i, 0)),
        out_shape=jax.ShapeDtypeStruct((N, D), jnp.float32),
    )(p, c, x, wlT, bl2, wrT, thresh)


def _tc_min_body(x_ref, o_ref):
    m = jnp.min(x_ref[...], axis=0, keepdims=True)
    i = pl.program_id(0)

    @pl.when(i == 0)
    def _():
        o_ref[...] = m

    @pl.when(i != 0)
    def _():
        o_ref[...] = jnp.minimum(o_ref[...], m)


def _tc_min(h):
    out = pl.pallas_call(
        _tc_min_body,
        grid=(N // R,),
        in_specs=[pl.BlockSpec((R, D), lambda i: (i, 0))],
        out_specs=pl.BlockSpec((1, D), lambda i: (0, 0)),
        out_shape=jax.ShapeDtypeStruct((1, D), jnp.float32),
    )(h)
    return out.reshape(D)


def kernel(x, edge_index, Wl0, bl0, Wr0, Wl1, bl1, Wr1):
    eidx = jnp.left_shift(edge_index[0], SHIFT) | edge_index[1]
    wlTs = jnp.stack([Wl0.T, Wl1.T])
    wrTs = jnp.stack([Wr0.T, Wr1.T])
    bls = jnp.stack([bl0.reshape(1, D), bl1.reshape(1, D)])
    ths = jnp.stack([jnp.zeros((1, 1), jnp.float32),
                     jnp.full((1, 1), -jnp.inf, jnp.float32)])

    def body(h, ws):
        wlT, bl2, wrT, th = ws
        p, c = _AGG(h, eidx)
        h2 = _tc_layer(p, c, h, wlT, bl2, wrT, th)
        return h2, None

    h_final, _ = lax.scan(body, x, (wlTs, bls, wrTs, ths))
    return _tc_min(h_final)


# speedup vs baseline: 9.8663x; 1.6211x over previous
import jax
import jax.numpy as jnp
from jax import lax
from jax.experimental import pallas as pl
from jax.experimental.pallas import tpu as pltpu
from jax.experimental.pallas import tpu_sc as plsc

N = 10000
E = 320000
D = 128
CW = 16
SHIFT = 14

NC = 2
NS = 16
NW = NC * NS
EPW = E // NW
C = 80
CHUNKS = EPW // C
RPT = 624
RLAST = N - (NS - 1) * RPT
WCH = 80


def _agg_body(x_hbm, eidx_hbm, outp_hbm, outc_hbm,
              idx_v, src_v, dst_v, rows_v, idx2_v, src2_v, dst2_v, rows2_v,
              ones_v, obuf_v, cbuf_v, acc_sh, cnt_sh, sem, sem2):
    cid = lax.axis_index("c")
    sid = lax.axis_index("s")
    wid = sid * NC + cid
    zbase = sid * RPT
    base_e = wid * EPW
    last = sid == NS - 1

    z16 = jnp.zeros((16,), jnp.float32)
    o16 = jnp.ones((16,), jnp.float32)

    @pl.loop(0, WCH)
    def _zrow(r):
        for j in range(D // 16):
            obuf_v[r, pl.ds(j * 16, 16)] = z16
        cbuf_v[r, :] = z16
        ones_v[r, :] = o16

    for k in range(7):
        pltpu.sync_copy(obuf_v, acc_sh.at[pl.ds(zbase + k * WCH, WCH)])
        pltpu.sync_copy(cbuf_v, cnt_sh.at[pl.ds(zbase + k * WCH, WCH)])

    @pl.when(~last)
    def _():
        pltpu.sync_copy(obuf_v.at[pl.ds(0, RPT - 7 * WCH)],
                        acc_sh.at[pl.ds(zbase + 7 * WCH, RPT - 7 * WCH)])
        pltpu.sync_copy(cbuf_v.at[pl.ds(0, RPT - 7 * WCH)],
                        cnt_sh.at[pl.ds(zbase + 7 * WCH, RPT - 7 * WCH)])

    @pl.when(last)
    def _():
        pltpu.sync_copy(obuf_v, acc_sh.at[pl.ds(zbase + 7 * WCH, WCH)])
        pltpu.sync_copy(cbuf_v, cnt_sh.at[pl.ds(zbase + 7 * WCH, WCH)])

    plsc.subcore_barrier()

    bufs = ((idx_v, src_v, dst_v, rows_v, sem),
            (idx2_v, src2_v, dst2_v, rows2_v, sem2))

    def fetch(i, b):
        idxb, srcb, dstb, _, _ = bufs[b]
        pltpu.sync_copy(eidx_hbm.at[pl.ds(base_e + i * C, C)], idxb)
        for k in range(C // 16):
            w = idxb[pl.ds(k * 16, 16)]
            srcb[pl.ds(k * 16, 16)] = lax.shift_right_logical(w, SHIFT)
            dstb[0, pl.ds(k * 16, 16)] = lax.bitwise_and(w, (1 << SHIFT) - 1)

    def gstart(b):
        _, srcb, _, rowsb, semb = bufs[b]
        pltpu.async_copy(x_hbm.at[srcb], rowsb, semb)

    def gwait(b):
        _, srcb, _, rowsb, semb = bufs[b]
        pltpu.make_async_copy(x_hbm.at[srcb], rowsb, semb).wait()

    def scat(b):
        _, _, dstb, rowsb, _ = bufs[b]
        pltpu.sync_copy(rowsb, acc_sh.at[dstb.at[0]], add=True)
        pltpu.sync_copy(ones_v, cnt_sh.at[dstb.at[0]], add=True)

    fetch(0, 0)
    gstart(0)

    @pl.loop(0, (CHUNKS - 1) // 2)
    def _pair(t):
        i = 2 * t
        fetch(i + 1, 1)
        gstart(1)
        gwait(0)
        scat(0)
        fetch(i + 2, 0)
        gstart(0)
        gwait(1)
        scat(1)

    gwait(0)
    scat(0)

    plsc.subcore_barrier()

    def out_chunk(rbase, nrows):
        pltpu.sync_copy(acc_sh.at[pl.ds(rbase, nrows)],
                        obuf_v.at[pl.ds(0, nrows)])
        pltpu.sync_copy(obuf_v.at[pl.ds(0, nrows)],
                        outp_hbm.at[cid, pl.ds(rbase, nrows)])
        pltpu.sync_copy(cnt_sh.at[pl.ds(rbase, nrows)],
                        cbuf_v.at[pl.ds(0, nrows)])
        pltpu.sync_copy(cbuf_v.at[pl.ds(0, nrows)],
                        outc_hbm.at[cid, pl.ds(rbase, nrows)])

    for k in range(7):
        out_chunk(zbase + k * WCH, WCH)

    @pl.when(~last)
    def _():
        out_chunk(zbase + 7 * WCH, RPT - 7 * WCH)

    @pl.when(last)
    def _():
        out_chunk(zbase + 7 * WCH, WCH)


_MESH = plsc.VectorSubcoreMesh(core_axis_name="c", subcore_axis_name="s")
_AGG = pl.kernel(
    _agg_body,
    out_type=[
        jax.ShapeDtypeStruct((NC, N, D), jnp.float32),
        jax.ShapeDtypeStruct((NC, N, CW), jnp.float32),
    ],
    mesh=_MESH,
    compiler_params=pltpu.CompilerParams(use_tc_tiling_on_sc=False),
    scratch_types=[
        pltpu.VMEM((C,), jnp.int32),
        pltpu.VMEM((C,), jnp.int32),
        pltpu.VMEM((1, C), jnp.int32),
        pltpu.VMEM((C, D), jnp.float32),
        pltpu.VMEM((C,), jnp.int32),
        pltpu.VMEM((C,), jnp.int32),
        pltpu.VMEM((1, C), jnp.int32),
        pltpu.VMEM((C, D), jnp.float32),
        pltpu.VMEM((WCH, CW), jnp.float32),
        pltpu.VMEM((WCH, D), jnp.float32),
        pltpu.VMEM((WCH, CW), jnp.float32),
        pltpu.VMEM_SHARED((N, D), jnp.float32),
        pltpu.VMEM_SHARED((N, CW), jnp.float32),
        pltpu.SemaphoreType.DMA,
        pltpu.SemaphoreType.DMA,
    ],
)


R = 2000


def _tc_layer_body(p_ref, c_ref, x_ref, wlT_ref, bl_ref, wrT_ref, th_ref,
                   o_ref):
    s = p_ref[0] + p_ref[1]
    cnt = c_ref[0, :, 0:1] + c_ref[1, :, 0:1]
    mean = s / jnp.maximum(cnt, 1.0)
    h = (jnp.dot(mean, wlT_ref[...], preferred_element_type=jnp.float32)
         + bl_ref[...]
         + jnp.dot(x_ref[...], wrT_ref[...], preferred_element_type=jnp.float32))
    o_ref[...] = jnp.maximum(h, th_ref[0, 0])


def _tc_layer(p, c, x, wlT, bl2, wrT, thresh):
    grid = (N // R,)
    in_specs = [
        pl.BlockSpec((NC, R, D), lambda i: (0, i, 0)),
        pl.BlockSpec((NC, R, CW), lambda i: (0, i, 0)),
        pl.BlockSpec((R, D), lambda i: (i, 0)),
        pl.BlockSpec((D, D), lambda i: (0, 0)),
        pl.BlockSpec((1, D), lambda i: (0, 0)),
        pl.BlockSpec((D, D), lambda i: (0, 0)),
        pl.BlockSpec((1, 1), lambda i: (0, 0)),
    ]
    return pl.pallas_call(
        _tc_layer_body,
        grid=grid,
        in_specs=in_specs,
        out_specs=pl.BlockSpec((R, D), lambda i: (i, 0)),
        out_shape=jax.ShapeDtypeStruct((N, D), jnp.float32),
    )(p, c, x, wlT, bl2, wrT, thresh)


def _tc_min_body(x_ref, o_ref):
    m = jnp.min(x_ref[...], axis=0, keepdims=True)
    i = pl.program_id(0)

    @pl.when(i == 0)
    def _():
        o_ref[...] = m

    @pl.when(i != 0)
    def _():
        o_ref[...] = jnp.minimum(o_ref[...], m)


def _tc_min(h):
    out = pl.pallas_call(
        _tc_min_body,
        grid=(N // R,),
        in_specs=[pl.BlockSpec((R, D), lambda i: (i, 0))],
        out_specs=pl.BlockSpec((1, D), lambda i: (0, 0)),
        out_shape=jax.ShapeDtypeStruct((1, D), jnp.float32),
    )(h)
    return out.reshape(D)


def kernel(x, edge_index, Wl0, bl0, Wr0, Wl1, bl1, Wr1):
    eidx = jnp.left_shift(edge_index[0], SHIFT) | edge_index[1]
    wlTs = jnp.stack([Wl0.T, Wl1.T])
    wrTs = jnp.stack([Wr0.T, Wr1.T])
    bls = jnp.stack([bl0.reshape(1, D), bl1.reshape(1, D)])
    ths = jnp.stack([jnp.zeros((1, 1), jnp.float32),
                     jnp.full((1, 1), -jnp.inf, jnp.float32)])

    def body(h, ws):
        wlT, bl2, wrT, th = ws
        p, c = _AGG(h, eidx)
        h2 = _tc_layer(p, c, h, wlT, bl2, wrT, th)
        return h2, None

    h_final, _ = lax.scan(body, x, (wlTs, bls, wrTs, ths))
    return _tc_min(h_final)


# speedup vs baseline: 9.8779x; 1.0012x over previous
import jax
import jax.numpy as jnp
from jax import lax
from jax.experimental import pallas as pl
from jax.experimental.pallas import tpu as pltpu
from jax.experimental.pallas import tpu_sc as plsc

N = 10000
E = 320000
D = 128
CW = 16
SHIFT = 14

NC = 2
NS = 16
NW = NC * NS
EPW = E // NW
C = 80
CHUNKS = EPW // C
RPT = 624
RLAST = N - (NS - 1) * RPT
WCH = 80


def _agg_body(x_hbm, eidx_hbm, outp_hbm, outc_hbm,
              idx_v, src_v, dst_v, rows_v, idx2_v, src2_v, dst2_v, rows2_v,
              ones_v, obuf_v, cbuf_v, acc_sh, cnt_sh, sem, sem2, ssem, ssem2):
    cid = lax.axis_index("c")
    sid = lax.axis_index("s")
    wid = sid * NC + cid
    zbase = sid * RPT
    base_e = wid * EPW
    last = sid == NS - 1

    z16 = jnp.zeros((16,), jnp.float32)
    o16 = jnp.ones((16,), jnp.float32)

    @pl.loop(0, WCH)
    def _zrow(r):
        for j in range(D // 16):
            obuf_v[r, pl.ds(j * 16, 16)] = z16
        cbuf_v[r, :] = z16
        ones_v[r, :] = o16

    for k in range(7):
        pltpu.sync_copy(obuf_v, acc_sh.at[pl.ds(zbase + k * WCH, WCH)])
        pltpu.sync_copy(cbuf_v, cnt_sh.at[pl.ds(zbase + k * WCH, WCH)])

    @pl.when(~last)
    def _():
        pltpu.sync_copy(obuf_v.at[pl.ds(0, RPT - 7 * WCH)],
                        acc_sh.at[pl.ds(zbase + 7 * WCH, RPT - 7 * WCH)])
        pltpu.sync_copy(cbuf_v.at[pl.ds(0, RPT - 7 * WCH)],
                        cnt_sh.at[pl.ds(zbase + 7 * WCH, RPT - 7 * WCH)])

    @pl.when(last)
    def _():
        pltpu.sync_copy(obuf_v, acc_sh.at[pl.ds(zbase + 7 * WCH, WCH)])
        pltpu.sync_copy(cbuf_v, cnt_sh.at[pl.ds(zbase + 7 * WCH, WCH)])

    plsc.subcore_barrier()

    bufs = ((idx_v, src_v, dst_v, rows_v, sem, ssem),
            (idx2_v, src2_v, dst2_v, rows2_v, sem2, ssem2))

    def fetch(i, b):
        idxb, srcb, dstb, _, _, _ = bufs[b]
        pltpu.sync_copy(eidx_hbm.at[pl.ds(base_e + i * C, C)], idxb)
        for k in range(C // 16):
            w = idxb[pl.ds(k * 16, 16)]
            srcb[pl.ds(k * 16, 16)] = lax.shift_right_logical(w, SHIFT)
            dstb[0, pl.ds(k * 16, 16)] = lax.bitwise_and(w, (1 << SHIFT) - 1)

    def gstart(b):
        _, srcb, _, rowsb, semb, _ = bufs[b]
        pltpu.async_copy(x_hbm.at[srcb], rowsb, semb)

    def gwait(b):
        _, srcb, _, rowsb, semb, _ = bufs[b]
        pltpu.make_async_copy(x_hbm.at[srcb], rowsb, semb).wait()

    def sstart(b):
        _, _, dstb, rowsb, _, ssemb = bufs[b]
        pltpu.async_copy(rowsb, acc_sh.at[dstb.at[0]], ssemb, add=True)
        pltpu.async_copy(ones_v, cnt_sh.at[dstb.at[0]], ssemb, add=True)

    def swait(b):
        _, _, dstb, rowsb, _, ssemb = bufs[b]
        pltpu.make_async_copy(rowsb, acc_sh.at[dstb.at[0]], ssemb).wait()
        pltpu.make_async_copy(ones_v, cnt_sh.at[dstb.at[0]], ssemb).wait()

    fetch(0, 0)
    gstart(0)

    @pl.loop(0, (CHUNKS - 1) // 2)
    def _pair(t):
        i = 2 * t

        @pl.when(t != 0)
        def _():
            swait(1)

        fetch(i + 1, 1)
        gstart(1)
        gwait(0)
        sstart(0)
        gwait(1)
        sstart(1)
        swait(0)
        fetch(i + 2, 0)
        gstart(0)

    gwait(0)
    sstart(0)
    swait(1)
    swait(0)

    plsc.subcore_barrier()

    def out_chunk(rbase, nrows):
        pltpu.sync_copy(acc_sh.at[pl.ds(rbase, nrows)],
                        obuf_v.at[pl.ds(0, nrows)])
        pltpu.sync_copy(obuf_v.at[pl.ds(0, nrows)],
                        outp_hbm.at[cid, pl.ds(rbase, nrows)])
        pltpu.sync_copy(cnt_sh.at[pl.ds(rbase, nrows)],
                        cbuf_v.at[pl.ds(0, nrows)])
        pltpu.sync_copy(cbuf_v.at[pl.ds(0, nrows)],
                        outc_hbm.at[cid, pl.ds(rbase, nrows)])

    for k in range(7):
        out_chunk(zbase + k * WCH, WCH)

    @pl.when(~last)
    def _():
        out_chunk(zbase + 7 * WCH, RPT - 7 * WCH)

    @pl.when(last)
    def _():
        out_chunk(zbase + 7 * WCH, WCH)


_MESH = plsc.VectorSubcoreMesh(core_axis_name="c", subcore_axis_name="s")
_AGG = pl.kernel(
    _agg_body,
    out_type=[
        jax.ShapeDtypeStruct((NC, N, D), jnp.float32),
        jax.ShapeDtypeStruct((NC, N, CW), jnp.float32),
    ],
    mesh=_MESH,
    compiler_params=pltpu.CompilerParams(use_tc_tiling_on_sc=False),
    scratch_types=[
        pltpu.VMEM((C,), jnp.int32),
        pltpu.VMEM((C,), jnp.int32),
        pltpu.VMEM((1, C), jnp.int32),
        pltpu.VMEM((C, D), jnp.float32),
        pltpu.VMEM((C,), jnp.int32),
        pltpu.VMEM((C,), jnp.int32),
        pltpu.VMEM((1, C), jnp.int32),
        pltpu.VMEM((C, D), jnp.float32),
        pltpu.VMEM((WCH, CW), jnp.float32),
        pltpu.VMEM((WCH, D), jnp.float32),
        pltpu.VMEM((WCH, CW), jnp.float32),
        pltpu.VMEM_SHARED((N, D), jnp.float32),
        pltpu.VMEM_SHARED((N, CW), jnp.float32),
        pltpu.SemaphoreType.DMA,
        pltpu.SemaphoreType.DMA,
        pltpu.SemaphoreType.DMA,
        pltpu.SemaphoreType.DMA,
    ],
)


R = 2000


def _tc_layer_body(p_ref, c_ref, x_ref, wlT_ref, bl_ref, wrT_ref, th_ref,
                   o_ref):
    s = p_ref[0] + p_ref[1]
    cnt = c_ref[0, :, 0:1] + c_ref[1, :, 0:1]
    mean = s / jnp.maximum(cnt, 1.0)
    h = (jnp.dot(mean, wlT_ref[...], preferred_element_type=jnp.float32)
         + bl_ref[...]
         + jnp.dot(x_ref[...], wrT_ref[...], preferred_element_type=jnp.float32))
    o_ref[...] = jnp.maximum(h, th_ref[0, 0])


def _tc_layer(p, c, x, wlT, bl2, wrT, thresh):
    grid = (N // R,)
    in_specs = [
        pl.BlockSpec((NC, R, D), lambda i: (0, i, 0)),
        pl.BlockSpec((NC, R, CW), lambda i: (0, i, 0)),
        pl.BlockSpec((R, D), lambda i: (i, 0)),
        pl.BlockSpec((D, D), lambda i: (0, 0)),
        pl.BlockSpec((1, D), lambda i: (0, 0)),
        pl.BlockSpec((D, D), lambda i: (0, 0)),
        pl.BlockSpec((1, 1), lambda i: (0, 0)),
    ]
    return pl.pallas_call(
        _tc_layer_body,
        grid=grid,
        in_specs=in_specs,
        out_specs=pl.BlockSpec((R, D), lambda i: (i, 0)),
        out_shape=jax.ShapeDtypeStruct((N, D), jnp.float32),
    )(p, c, x, wlT, bl2, wrT, thresh)


def _tc_min_body(x_ref, o_ref):
    m = jnp.min(x_ref[...], axis=0, keepdims=True)
    i = pl.program_id(0)

    @pl.when(i == 0)
    def _():
        o_ref[...] = m

    @pl.when(i != 0)
    def _():
        o_ref[...] = jnp.minimum(o_ref[...], m)


def _tc_min(h):
    out = pl.pallas_call(
        _tc_min_body,
        grid=(N // R,),
        in_specs=[pl.BlockSpec((R, D), lambda i: (i, 0))],
        out_specs=pl.BlockSpec((1, D), lambda i: (0, 0)),
        out_shape=jax.ShapeDtypeStruct((1, D), jnp.float32),
    )(h)
    return out.reshape(D)


def kernel(x, edge_index, Wl0, bl0, Wr0, Wl1, bl1, Wr1):
    eidx = jnp.left_shift(edge_index[0], SHIFT) | edge_index[1]
    wlTs = jnp.stack([Wl0.T, Wl1.T])
    wrTs = jnp.stack([Wr0.T, Wr1.T])
    bls = jnp.stack([bl0.reshape(1, D), bl1.reshape(1, D)])
    ths = jnp.stack([jnp.zeros((1, 1), jnp.float32),
                     jnp.full((1, 1), -jnp.inf, jnp.float32)])

    def body(h, ws):
        wlT, bl2, wrT, th = ws
        p, c = _AGG(h, eidx)
        h2 = _tc_layer(p, c, h, wlT, bl2, wrT, th)
        return h2, None

    h_final, _ = lax.scan(body, x, (wlTs, bls, wrTs, ths))
    return _tc_min(h_final)


# speedup vs baseline: 11.5391x; 1.1682x over previous
import jax
import jax.numpy as jnp
from jax import lax
from jax.experimental import pallas as pl
from jax.experimental.pallas import tpu as pltpu
from jax.experimental.pallas import tpu_sc as plsc

N = 10000
E = 320000
D = 128
CW = 16
SHIFT = 14

NC = 2
NS = 16
NW = NC * NS
EPW = E // NW
C = 80
CHUNKS = EPW // C
RPT = 624
RLAST = N - (NS - 1) * RPT
WCH = 80


def _agg_body(x_hbm, eidx_hbm, outp_hbm, outc_hbm,
              idx_v, src_v, dst_v, rows_v, idx2_v, src2_v, dst2_v, rows2_v,
              idx3_v, src3_v, dst3_v, rows3_v,
              ones_v, cbuf_v, acc_sh, cnt_sh,
              sem, sem2, sem3, ssem, ssem2, ssem3):
    obuf_v = rows3_v
    cid = lax.axis_index("c")
    sid = lax.axis_index("s")
    wid = sid * NC + cid
    zbase = sid * RPT
    base_e = wid * EPW
    last = sid == NS - 1

    z16 = jnp.zeros((16,), jnp.float32)
    o16 = jnp.ones((16,), jnp.float32)

    @pl.loop(0, WCH)
    def _zrow(r):
        for j in range(D // 16):
            obuf_v[r, pl.ds(j * 16, 16)] = z16
        cbuf_v[r, :] = z16
        ones_v[r, :] = o16

    for k in range(7):
        pltpu.sync_copy(obuf_v, acc_sh.at[pl.ds(zbase + k * WCH, WCH)])
        pltpu.sync_copy(cbuf_v, cnt_sh.at[pl.ds(zbase + k * WCH, WCH)])

    @pl.when(~last)
    def _():
        pltpu.sync_copy(obuf_v.at[pl.ds(0, RPT - 7 * WCH)],
                        acc_sh.at[pl.ds(zbase + 7 * WCH, RPT - 7 * WCH)])
        pltpu.sync_copy(cbuf_v.at[pl.ds(0, RPT - 7 * WCH)],
                        cnt_sh.at[pl.ds(zbase + 7 * WCH, RPT - 7 * WCH)])

    @pl.when(last)
    def _():
        pltpu.sync_copy(obuf_v, acc_sh.at[pl.ds(zbase + 7 * WCH, WCH)])
        pltpu.sync_copy(cbuf_v, cnt_sh.at[pl.ds(zbase + 7 * WCH, WCH)])

    plsc.subcore_barrier()

    bufs = ((idx_v, src_v, dst_v, rows_v, sem, ssem),
            (idx2_v, src2_v, dst2_v, rows2_v, sem2, ssem2),
            (idx3_v, src3_v, dst3_v, rows3_v, sem3, ssem3))

    def fetch(i, b):
        idxb, srcb, dstb, _, _, _ = bufs[b]
        pltpu.sync_copy(eidx_hbm.at[pl.ds(base_e + i * C, C)], idxb)
        for k in range(C // 16):
            w = idxb[pl.ds(k * 16, 16)]
            srcb[pl.ds(k * 16, 16)] = lax.shift_right_logical(w, SHIFT)
            dstb[0, pl.ds(k * 16, 16)] = lax.bitwise_and(w, (1 << SHIFT) - 1)

    def gstart(b):
        _, srcb, _, rowsb, semb, _ = bufs[b]
        pltpu.async_copy(x_hbm.at[srcb], rowsb, semb)

    def gwait(b):
        _, srcb, _, rowsb, semb, _ = bufs[b]
        pltpu.make_async_copy(x_hbm.at[srcb], rowsb, semb).wait()

    def sstart(b):
        _, _, dstb, rowsb, _, ssemb = bufs[b]
        pltpu.async_copy(rowsb, acc_sh.at[dstb.at[0]], ssemb, add=True)
        pltpu.async_copy(ones_v, cnt_sh.at[dstb.at[0]], ssemb, add=True)

    def swait(b):
        _, _, dstb, rowsb, _, ssemb = bufs[b]
        pltpu.make_async_copy(rowsb, acc_sh.at[dstb.at[0]], ssemb).wait()
        pltpu.make_async_copy(ones_v, cnt_sh.at[dstb.at[0]], ssemb).wait()

    fetch(0, 0)
    gstart(0)
    fetch(1, 1)
    gstart(1)

    T = (CHUNKS - 2) // 3

    @pl.loop(0, T)
    def _trip(t):
        i = 3 * t
        for k in range(3):
            j = i + k
            b = k % 3
            bf = (k + 2) % 3
            gwait(b)
            sstart(b)
            if k == 0:
                @pl.when(t != 0)
                def _():
                    swait(bf)
            else:
                swait(bf)
            fetch(j + 2, bf)
            gstart(bf)

    for j in range(3 * T, CHUNKS):
        b = j % 3
        gwait(b)
        sstart(b)
    for b in range(3):
        swait(b)

    plsc.subcore_barrier()

    def out_chunk(rbase, nrows):
        pltpu.sync_copy(acc_sh.at[pl.ds(rbase, nrows)],
                        obuf_v.at[pl.ds(0, nrows)])
        pltpu.sync_copy(obuf_v.at[pl.ds(0, nrows)],
                        outp_hbm.at[cid, pl.ds(rbase, nrows)])
        pltpu.sync_copy(cnt_sh.at[pl.ds(rbase, nrows)],
                        cbuf_v.at[pl.ds(0, nrows)])
        pltpu.sync_copy(cbuf_v.at[pl.ds(0, nrows)],
                        outc_hbm.at[cid, pl.ds(rbase, nrows)])

    for k in range(7):
        out_chunk(zbase + k * WCH, WCH)

    @pl.when(~last)
    def _():
        out_chunk(zbase + 7 * WCH, RPT - 7 * WCH)

    @pl.when(last)
    def _():
        out_chunk(zbase + 7 * WCH, WCH)


_MESH = plsc.VectorSubcoreMesh(core_axis_name="c", subcore_axis_name="s")
_AGG = pl.kernel(
    _agg_body,
    out_type=[
        jax.ShapeDtypeStruct((NC, N, D), jnp.float32),
        jax.ShapeDtypeStruct((NC, N, CW), jnp.float32),
    ],
    mesh=_MESH,
    compiler_params=pltpu.CompilerParams(use_tc_tiling_on_sc=False),
    scratch_types=[
        pltpu.VMEM((C,), jnp.int32),
        pltpu.VMEM((C,), jnp.int32),
        pltpu.VMEM((1, C), jnp.int32),
        pltpu.VMEM((C, D), jnp.float32),
        pltpu.VMEM((C,), jnp.int32),
        pltpu.VMEM((C,), jnp.int32),
        pltpu.VMEM((1, C), jnp.int32),
        pltpu.VMEM((C, D), jnp.float32),
        pltpu.VMEM((C,), jnp.int32),
        pltpu.VMEM((C,), jnp.int32),
        pltpu.VMEM((1, C), jnp.int32),
        pltpu.VMEM((C, D), jnp.float32),
        pltpu.VMEM((WCH, CW), jnp.float32),
        pltpu.VMEM((WCH, CW), jnp.float32),
        pltpu.VMEM_SHARED((N, D), jnp.float32),
        pltpu.VMEM_SHARED((N, CW), jnp.float32),
        pltpu.SemaphoreType.DMA,
        pltpu.SemaphoreType.DMA,
        pltpu.SemaphoreType.DMA,
        pltpu.SemaphoreType.DMA,
        pltpu.SemaphoreType.DMA,
        pltpu.SemaphoreType.DMA,
    ],
)


R = 2000


def _tc_layer_body(p_ref, c_ref, x_ref, wlT_ref, bl_ref, wrT_ref, th_ref,
                   o_ref):
    s = p_ref[0] + p_ref[1]
    cnt = c_ref[0, :, 0:1] + c_ref[1, :, 0:1]
    mean = s / jnp.maximum(cnt, 1.0)
    h = (jnp.dot(mean, wlT_ref[...], preferred_element_type=jnp.float32)
         + bl_ref[...]
         + jnp.dot(x_ref[...], wrT_ref[...], preferred_element_type=jnp.float32))
    o_ref[...] = jnp.maximum(h, th_ref[0, 0])


def _tc_layer(p, c, x, wlT, bl2, wrT, thresh):
    grid = (N // R,)
    in_specs = [
        pl.BlockSpec((NC, R, D), lambda i: (0, i, 0)),
        pl.BlockSpec((NC, R, CW), lambda i: (0, i, 0)),
        pl.BlockSpec((R, D), lambda i: (i, 0)),
        pl.BlockSpec((D, D), lambda i: (0, 0)),
        pl.BlockSpec((1, D), lambda i: (0, 0)),
        pl.BlockSpec((D, D), lambda i: (0, 0)),
        pl.BlockSpec((1, 1), lambda i: (0, 0)),
    ]
    return pl.pallas_call(
        _tc_layer_body,
        grid=grid,
        in_specs=in_specs,
        out_specs=pl.BlockSpec((R, D), lambda i: (i, 0)),
        out_shape=jax.ShapeDtypeStruct((N, D), jnp.float32),
    )(p, c, x, wlT, bl2, wrT, thresh)


def _tc_min_body(x_ref, o_ref):
    m = jnp.min(x_ref[...], axis=0, keepdims=True)
    i = pl.program_id(0)

    @pl.when(i == 0)
    def _():
        o_ref[...] = m

    @pl.when(i != 0)
    def _():
        o_ref[...] = jnp.minimum(o_ref[...], m)


def _tc_min(h):
    out = pl.pallas_call(
        _tc_min_body,
        grid=(N // R,),
        in_specs=[pl.BlockSpec((R, D), lambda i: (i, 0))],
        out_specs=pl.BlockSpec((1, D), lambda i: (0, 0)),
        out_shape=jax.ShapeDtypeStruct((1, D), jnp.float32),
    )(h)
    return out.reshape(D)


def kernel(x, edge_index, Wl0, bl0, Wr0, Wl1, bl1, Wr1):
    eidx = jnp.left_shift(edge_index[0], SHIFT) | edge_index[1]
    wlTs = jnp.stack([Wl0.T, Wl1.T])
    wrTs = jnp.stack([Wr0.T, Wr1.T])
    bls = jnp.stack([bl0.reshape(1, D), bl1.reshape(1, D)])
    ths = jnp.stack([jnp.zeros((1, 1), jnp.float32),
                     jnp.full((1, 1), -jnp.inf, jnp.float32)])

    def body(h, ws):
        wlT, bl2, wrT, th = ws
        p, c = _AGG(h, eidx)
        h2 = _tc_layer(p, c, h, wlT, bl2, wrT, th)
        return h2, None

    h_final, _ = lax.scan(body, x, (wlTs, bls, wrTs, ths))
    return _tc_min(h_final)


# speedup vs baseline: 11.7267x; 1.0163x over previous
import jax
import jax.numpy as jnp
from jax import lax
from jax.experimental import pallas as pl
from jax.experimental.pallas import tpu as pltpu
from jax.experimental.pallas import tpu_sc as plsc

N = 10000
E = 320000
D = 128
CW = 16
SHIFT = 14

NC = 2
NS = 16
NW = NC * NS
EPW = E // NW
C = 80
CHUNKS = EPW // C
RPT = 624
RLAST = N - (NS - 1) * RPT
WCH = 80


def _agg_body(x_hbm, eidx_hbm, outp_hbm, outc_hbm,
              idx_v, src_v, dst_v, rows_v, idx2_v, src2_v, dst2_v, rows2_v,
              idx3_v, src3_v, dst3_v, rows3_v,
              ones_v, cbuf_v, acc_sh, cnt_sh,
              sem, sem2, sem3, ssem, ssem2, ssem3):
    obuf_v = rows3_v
    cid = lax.axis_index("c")
    sid = lax.axis_index("s")
    wid = sid * NC + cid
    zbase = sid * RPT
    base_e = wid * EPW
    last = sid == NS - 1

    z16 = jnp.zeros((16,), jnp.float32)
    o16 = jnp.ones((16,), jnp.float32)

    @pl.loop(0, WCH)
    def _zrow(r):
        for j in range(D // 16):
            obuf_v[r, pl.ds(j * 16, 16)] = z16
        cbuf_v[r, :] = z16
        ones_v[r, :] = o16

    for k in range(7):
        pltpu.sync_copy(obuf_v, acc_sh.at[pl.ds(zbase + k * WCH, WCH)])
        pltpu.sync_copy(cbuf_v, cnt_sh.at[pl.ds(zbase + k * WCH, WCH)])

    @pl.when(~last)
    def _():
        pltpu.sync_copy(obuf_v.at[pl.ds(0, RPT - 7 * WCH)],
                        acc_sh.at[pl.ds(zbase + 7 * WCH, RPT - 7 * WCH)])
        pltpu.sync_copy(cbuf_v.at[pl.ds(0, RPT - 7 * WCH)],
                        cnt_sh.at[pl.ds(zbase + 7 * WCH, RPT - 7 * WCH)])

    @pl.when(last)
    def _():
        pltpu.sync_copy(obuf_v, acc_sh.at[pl.ds(zbase + 7 * WCH, WCH)])
        pltpu.sync_copy(cbuf_v, cnt_sh.at[pl.ds(zbase + 7 * WCH, WCH)])

    plsc.subcore_barrier()

    bufs = ((idx_v, src_v, dst_v, rows_v, sem, ssem),
            (idx2_v, src2_v, dst2_v, rows2_v, sem2, ssem2),
            (idx3_v, src3_v, dst3_v, rows3_v, sem3, ssem3))

    def fetch(i, b):
        idxb, srcb, dstb, _, _, _ = bufs[b]
        pltpu.sync_copy(eidx_hbm.at[pl.ds(base_e + i * C, C)], idxb)
        for k in range(C // 16):
            w = idxb[pl.ds(k * 16, 16)]
            srcb[pl.ds(k * 16, 16)] = lax.shift_right_logical(w, SHIFT)
            dstb[0, pl.ds(k * 16, 16)] = lax.bitwise_and(w, (1 << SHIFT) - 1)

    def gstart(b):
        _, srcb, _, rowsb, semb, _ = bufs[b]
        pltpu.async_copy(x_hbm.at[srcb], rowsb, semb)

    def gwait(b):
        _, srcb, _, rowsb, semb, _ = bufs[b]
        pltpu.make_async_copy(x_hbm.at[srcb], rowsb, semb).wait()

    def sstart(b):
        _, _, dstb, rowsb, _, ssemb = bufs[b]
        pltpu.async_copy(rowsb, acc_sh.at[dstb.at[0]], ssemb, add=True)
        pltpu.async_copy(ones_v, cnt_sh.at[dstb.at[0]], ssemb, add=True)

    def swait(b):
        _, _, dstb, rowsb, _, ssemb = bufs[b]
        pltpu.make_async_copy(rowsb, acc_sh.at[dstb.at[0]], ssemb).wait()
        pltpu.make_async_copy(ones_v, cnt_sh.at[dstb.at[0]], ssemb).wait()

    fetch(0, 0)
    gstart(0)
    fetch(1, 1)
    gstart(1)

    T = (CHUNKS - 2) // 3

    @pl.loop(0, T)
    def _trip(t):
        i = 3 * t
        for k in range(3):
            j = i + k
            b = k % 3
            bf = (k + 2) % 3
            gwait(b)
            sstart(b)
            if k == 0:
                @pl.when(t != 0)
                def _():
                    swait(bf)
            else:
                swait(bf)
            fetch(j + 2, bf)
            gstart(bf)

    for j in range(3 * T, CHUNKS):
        b = j % 3
        gwait(b)
        sstart(b)
    for b in range(3):
        swait(b)

    plsc.subcore_barrier()

    def out_chunk(rbase, nrows):
        pltpu.sync_copy(acc_sh.at[pl.ds(rbase, nrows)],
                        obuf_v.at[pl.ds(0, nrows)])
        pltpu.sync_copy(obuf_v.at[pl.ds(0, nrows)],
                        outp_hbm.at[cid, pl.ds(rbase, nrows)])
        pltpu.sync_copy(cnt_sh.at[pl.ds(rbase, nrows)],
                        cbuf_v.at[pl.ds(0, nrows)])
        pltpu.sync_copy(cbuf_v.at[pl.ds(0, nrows)],
                        outc_hbm.at[cid, pl.ds(rbase, nrows)])

    for k in range(7):
        out_chunk(zbase + k * WCH, WCH)

    @pl.when(~last)
    def _():
        out_chunk(zbase + 7 * WCH, RPT - 7 * WCH)

    @pl.when(last)
    def _():
        out_chunk(zbase + 7 * WCH, WCH)


_MESH = plsc.VectorSubcoreMesh(core_axis_name="c", subcore_axis_name="s")
_AGG = pl.kernel(
    _agg_body,
    out_type=[
        jax.ShapeDtypeStruct((NC, N, D), jnp.float32),
        jax.ShapeDtypeStruct((NC, N, CW), jnp.float32),
    ],
    mesh=_MESH,
    compiler_params=pltpu.CompilerParams(use_tc_tiling_on_sc=False),
    scratch_types=[
        pltpu.VMEM((C,), jnp.int32),
        pltpu.VMEM((C,), jnp.int32),
        pltpu.VMEM((1, C), jnp.int32),
        pltpu.VMEM((C, D), jnp.float32),
        pltpu.VMEM((C,), jnp.int32),
        pltpu.VMEM((C,), jnp.int32),
        pltpu.VMEM((1, C), jnp.int32),
        pltpu.VMEM((C, D), jnp.float32),
        pltpu.VMEM((C,), jnp.int32),
        pltpu.VMEM((C,), jnp.int32),
        pltpu.VMEM((1, C), jnp.int32),
        pltpu.VMEM((C, D), jnp.float32),
        pltpu.VMEM((WCH, CW), jnp.float32),
        pltpu.VMEM((WCH, CW), jnp.float32),
        pltpu.VMEM_SHARED((N, D), jnp.float32),
        pltpu.VMEM_SHARED((N, CW), jnp.float32),
        pltpu.SemaphoreType.DMA,
        pltpu.SemaphoreType.DMA,
        pltpu.SemaphoreType.DMA,
        pltpu.SemaphoreType.DMA,
        pltpu.SemaphoreType.DMA,
        pltpu.SemaphoreType.DMA,
    ],
)


R = 2000


def _tc_layer_body(p_ref, c_ref, x_ref, wlT_ref, bl_ref, wrT_ref, th_ref,
                   o_ref, m_ref):
    s = p_ref[0] + p_ref[1]
    cnt = c_ref[0, :, 0:1] + c_ref[1, :, 0:1]
    mean = s / jnp.maximum(cnt, 1.0)
    h = (jnp.dot(mean, wlT_ref[...], preferred_element_type=jnp.float32)
         + bl_ref[...]
         + jnp.dot(x_ref[...], wrT_ref[...], preferred_element_type=jnp.float32))
    h = jnp.maximum(h, th_ref[0, 0])
    o_ref[...] = h
    m = jnp.min(h, axis=0, keepdims=True)
    i = pl.program_id(0)

    @pl.when(i == 0)
    def _():
        m_ref[...] = m

    @pl.when(i != 0)
    def _():
        m_ref[...] = jnp.minimum(m_ref[...], m)


def _tc_layer(p, c, x, wlT, bl2, wrT, thresh):
    grid = (N // R,)
    in_specs = [
        pl.BlockSpec((NC, R, D), lambda i: (0, i, 0)),
        pl.BlockSpec((NC, R, CW), lambda i: (0, i, 0)),
        pl.BlockSpec((R, D), lambda i: (i, 0)),
        pl.BlockSpec((D, D), lambda i: (0, 0)),
        pl.BlockSpec((1, D), lambda i: (0, 0)),
        pl.BlockSpec((D, D), lambda i: (0, 0)),
        pl.BlockSpec((1, 1), lambda i: (0, 0)),
    ]
    return pl.pallas_call(
        _tc_layer_body,
        grid=grid,
        in_specs=in_specs,
        out_specs=[pl.BlockSpec((R, D), lambda i: (i, 0)),
                   pl.BlockSpec((1, D), lambda i: (0, 0))],
        out_shape=[jax.ShapeDtypeStruct((N, D), jnp.float32),
                   jax.ShapeDtypeStruct((1, D), jnp.float32)],
    )(p, c, x, wlT, bl2, wrT, thresh)


def kernel(x, edge_index, Wl0, bl0, Wr0, Wl1, bl1, Wr1):
    eidx = jnp.left_shift(edge_index[0], SHIFT) | edge_index[1]
    wlTs = jnp.stack([Wl0.T, Wl1.T])
    wrTs = jnp.stack([Wr0.T, Wr1.T])
    bls = jnp.stack([bl0.reshape(1, D), bl1.reshape(1, D)])
    ths = jnp.stack([jnp.zeros((1, 1), jnp.float32),
                     jnp.full((1, 1), -jnp.inf, jnp.float32)])

    def body(h, ws):
        wlT, bl2, wrT, th = ws
        p, c = _AGG(h, eidx)
        h2, m = _tc_layer(p, c, h, wlT, bl2, wrT, th)
        return h2, m

    _, ms = lax.scan(body, x, (wlTs, bls, wrTs, ths))
    return ms[1].reshape(D)


# speedup vs baseline: 13.3642x; 1.1396x over previous
import jax
import jax.numpy as jnp
from jax import lax
from jax.experimental import pallas as pl
from jax.experimental.pallas import tpu as pltpu
from jax.experimental.pallas import tpu_sc as plsc

N = 10000
E = 320000
D = 128
CW = 16
SHIFT = 14

NC = 2
NS = 16
NW = NC * NS
EPW = E // NW
C = 80
CHUNKS = EPW // C
RPT = 624
RLAST = N - (NS - 1) * RPT
WCH = 80


def _agg_body(x_hbm, eidx_hbm, outp_hbm, outc_hbm,
              idx_v, src_v, dst_v, rows_v, idx2_v, src2_v, dst2_v, rows2_v,
              idx3_v, src3_v, dst3_v, rows3_v,
              ones_v, cbuf_v, acc_sh, cnt_sh,
              sem, sem2, sem3, ssem, ssem2, ssem3, isem, isem2, isem3):
    obuf_v = rows3_v
    cid = lax.axis_index("c")
    sid = lax.axis_index("s")
    wid = sid * NC + cid
    zbase = sid * RPT
    base_e = wid * EPW
    last = sid == NS - 1

    z16 = jnp.zeros((16,), jnp.float32)
    o16 = jnp.ones((16,), jnp.float32)

    @pl.loop(0, WCH)
    def _zrow(r):
        for j in range(D // 16):
            obuf_v[r, pl.ds(j * 16, 16)] = z16
        cbuf_v[r, :] = z16
        ones_v[r, :] = o16

    for k in range(7):
        pltpu.sync_copy(obuf_v, acc_sh.at[pl.ds(zbase + k * WCH, WCH)])
        pltpu.sync_copy(cbuf_v, cnt_sh.at[pl.ds(zbase + k * WCH, WCH)])

    @pl.when(~last)
    def _():
        pltpu.sync_copy(obuf_v.at[pl.ds(0, RPT - 7 * WCH)],
                        acc_sh.at[pl.ds(zbase + 7 * WCH, RPT - 7 * WCH)])
        pltpu.sync_copy(cbuf_v.at[pl.ds(0, RPT - 7 * WCH)],
                        cnt_sh.at[pl.ds(zbase + 7 * WCH, RPT - 7 * WCH)])

    @pl.when(last)
    def _():
        pltpu.sync_copy(obuf_v, acc_sh.at[pl.ds(zbase + 7 * WCH, WCH)])
        pltpu.sync_copy(cbuf_v, cnt_sh.at[pl.ds(zbase + 7 * WCH, WCH)])

    plsc.subcore_barrier()

    bufs = ((idx_v, src_v, dst_v, rows_v, sem, ssem, isem),
            (idx2_v, src2_v, dst2_v, rows2_v, sem2, ssem2, isem2),
            (idx3_v, src3_v, dst3_v, rows3_v, sem3, ssem3, isem3))

    def ifetch(i, b):
        idxb, _, _, _, _, _, isemb = bufs[b]
        pltpu.async_copy(eidx_hbm.at[pl.ds(base_e + i * C, C)], idxb, isemb)

    def unpack(b):
        idxb, srcb, dstb, _, _, _, isemb = bufs[b]
        pltpu.make_async_copy(eidx_hbm.at[pl.ds(base_e, C)], idxb, isemb).wait()
        for k in range(C // 16):
            w = idxb[pl.ds(k * 16, 16)]
            srcb[pl.ds(k * 16, 16)] = lax.shift_right_logical(w, SHIFT)
            dstb[0, pl.ds(k * 16, 16)] = lax.bitwise_and(w, (1 << SHIFT) - 1)

    def gstart(b):
        _, srcb, _, rowsb, semb, _, _ = bufs[b]
        pltpu.async_copy(x_hbm.at[srcb], rowsb, semb)

    def gwait(b):
        _, srcb, _, rowsb, semb, _, _ = bufs[b]
        pltpu.make_async_copy(x_hbm.at[srcb], rowsb, semb).wait()

    def sstart(b):
        _, _, dstb, rowsb, _, ssemb, _ = bufs[b]
        pltpu.async_copy(rowsb, acc_sh.at[dstb.at[0]], ssemb, add=True)
        pltpu.async_copy(ones_v, cnt_sh.at[dstb.at[0]], ssemb, add=True)

    def swait(b):
        _, _, dstb, rowsb, _, ssemb, _ = bufs[b]
        pltpu.make_async_copy(rowsb, acc_sh.at[dstb.at[0]], ssemb).wait()
        pltpu.make_async_copy(ones_v, cnt_sh.at[dstb.at[0]], ssemb).wait()

    ifetch(0, 0)
    ifetch(1, 1)
    ifetch(2, 2)
    unpack(0)
    gstart(0)
    unpack(1)
    gstart(1)

    T = 40

    @pl.loop(0, T)
    def _trip(t):
        i = 3 * t
        for k in range(3):
            j = i + k
            b = k % 3
            bf = (k + 2) % 3
            gwait(b)
            sstart(b)
            if k == 0:
                @pl.when(t != 0)
                def _():
                    swait(bf)
            else:
                swait(bf)
            unpack(bf)
            gstart(bf)
            ifetch(j + 3, b)

    for j in range(120, CHUNKS):
        b = j % 3
        bf = (j + 2) % 3
        gwait(b)
        sstart(b)
        swait(bf)
        if j + 2 < CHUNKS:
            unpack(bf)
            gstart(bf)
        if j + 3 < CHUNKS:
            ifetch(j + 3, (j + 3) % 3)
    swait((CHUNKS - 1) % 3)

    plsc.subcore_barrier()

    def out_chunk(rbase, nrows):
        pltpu.sync_copy(acc_sh.at[pl.ds(rbase, nrows)],
                        obuf_v.at[pl.ds(0, nrows)])
        pltpu.sync_copy(obuf_v.at[pl.ds(0, nrows)],
                        outp_hbm.at[cid, pl.ds(rbase, nrows)])
        pltpu.sync_copy(cnt_sh.at[pl.ds(rbase, nrows)],
                        cbuf_v.at[pl.ds(0, nrows)])
        pltpu.sync_copy(cbuf_v.at[pl.ds(0, nrows)],
                        outc_hbm.at[cid, pl.ds(rbase, nrows)])

    for k in range(7):
        out_chunk(zbase + k * WCH, WCH)

    @pl.when(~last)
    def _():
        out_chunk(zbase + 7 * WCH, RPT - 7 * WCH)

    @pl.when(last)
    def _():
        out_chunk(zbase + 7 * WCH, WCH)


_MESH = plsc.VectorSubcoreMesh(core_axis_name="c", subcore_axis_name="s")
_AGG = pl.kernel(
    _agg_body,
    out_type=[
        jax.ShapeDtypeStruct((NC, N, D), jnp.float32),
        jax.ShapeDtypeStruct((NC, N, CW), jnp.float32),
    ],
    mesh=_MESH,
    compiler_params=pltpu.CompilerParams(use_tc_tiling_on_sc=False),
    scratch_types=[
        pltpu.VMEM((C,), jnp.int32),
        pltpu.VMEM((C,), jnp.int32),
        pltpu.VMEM((1, C), jnp.int32),
        pltpu.VMEM((C, D), jnp.float32),
        pltpu.VMEM((C,), jnp.int32),
        pltpu.VMEM((C,), jnp.int32),
        pltpu.VMEM((1, C), jnp.int32),
        pltpu.VMEM((C, D), jnp.float32),
        pltpu.VMEM((C,), jnp.int32),
        pltpu.VMEM((C,), jnp.int32),
        pltpu.VMEM((1, C), jnp.int32),
        pltpu.VMEM((C, D), jnp.float32),
        pltpu.VMEM((WCH, CW), jnp.float32),
        pltpu.VMEM((WCH, CW), jnp.float32),
        pltpu.VMEM_SHARED((N, D), jnp.float32),
        pltpu.VMEM_SHARED((N, CW), jnp.float32),
        pltpu.SemaphoreType.DMA,
        pltpu.SemaphoreType.DMA,
        pltpu.SemaphoreType.DMA,
        pltpu.SemaphoreType.DMA,
        pltpu.SemaphoreType.DMA,
        pltpu.SemaphoreType.DMA,
        pltpu.SemaphoreType.DMA,
        pltpu.SemaphoreType.DMA,
        pltpu.SemaphoreType.DMA,
    ],
)


R = 2000


def _tc_layer_body(p_ref, c_ref, x_ref, wlT_ref, bl_ref, wrT_ref, th_ref,
                   o_ref, m_ref):
    s = p_ref[0] + p_ref[1]
    cnt = c_ref[0, :, 0:1] + c_ref[1, :, 0:1]
    mean = s / jnp.maximum(cnt, 1.0)
    h = (jnp.dot(mean, wlT_ref[...], preferred_element_type=jnp.float32)
         + bl_ref[...]
         + jnp.dot(x_ref[...], wrT_ref[...], preferred_element_type=jnp.float32))
    h = jnp.maximum(h, th_ref[0, 0])
    o_ref[...] = h
    m = jnp.min(h, axis=0, keepdims=True)
    i = pl.program_id(0)

    @pl.when(i == 0)
    def _():
        m_ref[...] = m

    @pl.when(i != 0)
    def _():
        m_ref[...] = jnp.minimum(m_ref[...], m)


def _tc_layer(p, c, x, wlT, bl2, wrT, thresh):
    grid = (N // R,)
    in_specs = [
        pl.BlockSpec((NC, R, D), lambda i: (0, i, 0)),
        pl.BlockSpec((NC, R, CW), lambda i: (0, i, 0)),
        pl.BlockSpec((R, D), lambda i: (i, 0)),
        pl.BlockSpec((D, D), lambda i: (0, 0)),
        pl.BlockSpec((1, D), lambda i: (0, 0)),
        pl.BlockSpec((D, D), lambda i: (0, 0)),
        pl.BlockSpec((1, 1), lambda i: (0, 0)),
    ]
    return pl.pallas_call(
        _tc_layer_body,
        grid=grid,
        in_specs=in_specs,
        out_specs=[pl.BlockSpec((R, D), lambda i: (i, 0)),
                   pl.BlockSpec((1, D), lambda i: (0, 0))],
        out_shape=[jax.ShapeDtypeStruct((N, D), jnp.float32),
                   jax.ShapeDtypeStruct((1, D), jnp.float32)],
    )(p, c, x, wlT, bl2, wrT, thresh)


def kernel(x, edge_index, Wl0, bl0, Wr0, Wl1, bl1, Wr1):
    eidx = jnp.left_shift(edge_index[0], SHIFT) | edge_index[1]
    wlTs = jnp.stack([Wl0.T, Wl1.T])
    wrTs = jnp.stack([Wr0.T, Wr1.T])
    bls = jnp.stack([bl0.reshape(1, D), bl1.reshape(1, D)])
    ths = jnp.stack([jnp.zeros((1, 1), jnp.float32),
                     jnp.full((1, 1), -jnp.inf, jnp.float32)])

    def body(h, ws):
        wlT, bl2, wrT, th = ws
        p, c = _AGG(h, eidx)
        h2, m = _tc_layer(p, c, h, wlT, bl2, wrT, th)
        return h2, m

    _, ms = lax.scan(body, x, (wlTs, bls, wrTs, ths))
    return ms[1].reshape(D)
